# baseline (speedup 1.0000x reference)
import functools

import jax
import jax.numpy as jnp
from jax import lax
from jax.experimental import pallas as pl
from jax.experimental.pallas import tpu as pltpu
from jax.experimental.pallas import tpu_sc as plsc

B = 16384
D = 64
C = 20
N = 100
K = C + N
VOCAB = 1000000
RBLK = 128
NVREG = D // 16
DEPTH = 4
TRBW = 2048


def _tc_linearize(table):
  t_t = table.T

  def body(x_ref, o_ref):
    y = x_ref[...].T.reshape(TRBW // 2, 2, D)
    o_ref[:, :D] = y[:, 0, :]
    o_ref[:, D:] = y[:, 1, :]

  grid = (VOCAB + TRBW - 1) // TRBW
  out = pl.pallas_call(
      body,
      grid=(grid,),
      in_specs=[pl.BlockSpec((D, TRBW), lambda j: (0, j))],
      out_specs=pl.BlockSpec((TRBW // 2, 2 * D), lambda j: (j, 0)),
      out_shape=jax.ShapeDtypeStruct((VOCAB // 2, 2 * D), jnp.float32),
  )(t_t)
  return out.reshape(VOCAB, D)


@functools.lru_cache(maxsize=None)
def _build_sc_main():
  info = plsc.get_sparse_core_info()
  nc, ns = info.num_cores, info.num_subcores
  nw = nc * ns
  rpw = B // nw
  nblk = rpw // RBLK
  mesh = plsc.VectorSubcoreMesh(core_axis_name="c", subcore_axis_name="s")

  scratch = (
      pltpu.VMEM((RBLK, K), jnp.int32),
      [pltpu.VMEM((K, D), jnp.float32) for _ in range(DEPTH)],
      pltpu.VMEM((RBLK, D), jnp.float32),
      pltpu.VMEM((RBLK, D), jnp.float32),
      [pltpu.SemaphoreType.DMA for _ in range(DEPTH)],
  )

  @functools.partial(
      pl.kernel,
      out_type=(
          jax.ShapeDtypeStruct((B, D), jnp.float32),
          jax.ShapeDtypeStruct((B, D), jnp.float32),
      ),
      mesh=mesh,
      compiler_params=pltpu.CompilerParams(use_tc_tiling_on_sc=False),
      scratch_types=scratch,
  )
  def sc_main(idx_hbm, emb_hbm, ctxsum_o, negsum_o,
              idx_v, bufs, ctxsum_v, negsum_v, sems):
    wid = lax.axis_index("s") * nc + lax.axis_index("c")

    def issue(b, slot):
      pltpu.make_async_copy(
          emb_hbm.at[idx_v.at[b]], bufs[slot], sems[slot]).start()

    def drain(slot):
      pltpu.make_async_copy(
          emb_hbm.at[idx_v.at[0]], bufs[slot], sems[slot]).wait()

    def seg_sum(buf, lo, hi):
      sls = [pl.ds(16 * k, 16) for k in range(NVREG)]
      acc = [buf[lo, sls[k]] for k in range(NVREG)]
      unroll = 8
      n = hi - lo - 1
      rem = n % unroll

      def step(i, a):
        j0 = lo + 1 + i * unroll
        for u in range(unroll):
          a = [a[k] + buf[j0 + u, sls[k]] for k in range(NVREG)]
        return a

      acc = lax.fori_loop(0, n // unroll, step, acc)
      for j in range(hi - rem, hi):
        acc = [acc[k] + buf[j, sls[k]] for k in range(NVREG)]
      return acc

    def reduce_store(slot, b):
      buf = bufs[slot]
      acc_c = seg_sum(buf, 0, C)
      acc_n = seg_sum(buf, C, K)
      for k in range(NVREG):
        sl = pl.ds(16 * k, 16)
        ctxsum_v[b, sl] = acc_c[k]
        negsum_v[b, sl] = acc_n[k]

    def block(blk, carry):
      base = pl.multiple_of(wid * rpw + blk * RBLK, RBLK)
      pltpu.sync_copy(idx_hbm.at[pl.ds(base, RBLK), :], idx_v)

      for s in range(DEPTH - 1):
        issue(s, s)

      def group(g, c2):
        for s in range(DEPTH):
          b = g * DEPTH + s

          @pl.when(b + DEPTH - 1 < RBLK)
          def _():
            issue(b + DEPTH - 1, (s + DEPTH - 1) % DEPTH)

          drain(s)
          reduce_store(s, b)
        return c2

      lax.fori_loop(0, RBLK // DEPTH, group, 0)

      pltpu.sync_copy(ctxsum_v, ctxsum_o.at[pl.ds(base, RBLK), :])
      pltpu.sync_copy(negsum_v, negsum_o.at[pl.ds(base, RBLK), :])
      return carry

    lax.fori_loop(0, nblk, block, 0)

  return sc_main


@functools.lru_cache(maxsize=None)
def _build_sc_center():
  info = plsc.get_sparse_core_info()
  nc, ns = info.num_cores, info.num_subcores
  nw = nc * ns
  rpw = B // nw
  ngat = rpw // 128
  mesh = plsc.VectorSubcoreMesh(core_axis_name="c", subcore_axis_name="s")

  @functools.partial(
      pl.kernel,
      out_type=jax.ShapeDtypeStruct((B, D), jnp.float32),
      mesh=mesh,
      compiler_params=pltpu.CompilerParams(use_tc_tiling_on_sc=False),
      scratch_types=(
          pltpu.VMEM((B // 32,), jnp.int32),
          pltpu.VMEM((B // 32, D), jnp.float32),
          pltpu.SemaphoreType.DMA,
      ),
  )
  def sc_center(center_hbm, emb_hbm, crows_o, cidx_v, crows_v, sem):
    wid = lax.axis_index("s") * nc + lax.axis_index("c")
    base = pl.multiple_of(wid * rpw, rpw)
    pltpu.sync_copy(center_hbm.at[pl.ds(base, rpw)], cidx_v)
    for i in range(ngat):
      pltpu.make_async_copy(
          emb_hbm.at[cidx_v.at[pl.ds(i * 128, 128)]],
          crows_v.at[pl.ds(i * 128, 128), :], sem).start()
    for i in range(ngat):
      pltpu.make_async_copy(
          emb_hbm.at[cidx_v.at[pl.ds(0, 128)]],
          crows_v.at[pl.ds(0, 128), :], sem).wait()
    pltpu.sync_copy(crows_v, crows_o.at[pl.ds(base, rpw), :])

  return sc_center


def _tc_score(crows, ctxsum, negsum):
  bt = 2048

  def body(c_ref, cs_ref, ns_ref, o_ref):
    c = c_ref[...]
    s_ctx = jnp.sum(cs_ref[...] * c, axis=1)
    s_neg = jnp.sum(ns_ref[...] * c, axis=1)
    o_ref[...] = -(jax.nn.log_sigmoid(s_ctx) + jax.nn.log_sigmoid(-s_neg))

  return pl.pallas_call(
      body,
      grid=(B // bt,),
      in_specs=[pl.BlockSpec((bt, D), lambda i: (i, 0))] * 3,
      out_specs=pl.BlockSpec((bt,), lambda i: (i,)),
      out_shape=jax.ShapeDtypeStruct((B,), jnp.float32),
  )(crows, ctxsum, negsum)


def kernel(center_word_label, context_words_labels, neg_words_labels,
           input_embedding, output_embedding):
  idx_all = jnp.concatenate(
      [context_words_labels.astype(jnp.int32),
       neg_words_labels.astype(jnp.int32)], axis=1)
  out_lin = _tc_linearize(output_embedding)
  in_lin = _tc_linearize(input_embedding)
  ctxsum, negsum = _build_sc_main()(idx_all, out_lin)
  crows = _build_sc_center()(center_word_label.astype(jnp.int32), in_lin)
  return _tc_score(crows, ctxsum, negsum)


# speedup vs baseline: 1.6517x; 1.6517x over previous
import functools

import jax
import jax.numpy as jnp
from jax import lax
from jax.experimental import pallas as pl
from jax.experimental.pallas import tpu as pltpu
from jax.experimental.pallas import tpu_sc as plsc

B = 16384
D = 64
C = 20
N = 100
K = C + N
VOCAB = 1000000
RBLK = 128
NVREG = D // 16
DEPTH = 4
TRBW = 2048


HALF = TRBW // 2
NLIN = (VOCAB + TRBW - 1) // TRBW


def _tc_linearize(table):
  t_t = table.T

  def body(x_ref, o_ref):
    x = x_ref[...]
    o_ref[...] = jax.lax.concatenate([x[:, :HALF].T, x[:, HALF:].T], 1)

  out = pl.pallas_call(
      body,
      grid=(NLIN,),
      in_specs=[pl.BlockSpec((D, TRBW), lambda j: (0, j))],
      out_specs=pl.BlockSpec((HALF, 2 * D), lambda j: (j, 0)),
      out_shape=jax.ShapeDtypeStruct((NLIN * HALF, 2 * D), jnp.float32),
  )(t_t)
  return out.reshape(NLIN * TRBW, D)


def _remap_idx(r):
  j = r // TRBW
  t = r % TRBW
  return 2 * (j * HALF + (t % HALF)) + t // HALF


@functools.lru_cache(maxsize=None)
def _build_sc_main():
  info = plsc.get_sparse_core_info()
  nc, ns = info.num_cores, info.num_subcores
  nw = nc * ns
  rpw = B // nw
  nblk = rpw // RBLK
  mesh = plsc.VectorSubcoreMesh(core_axis_name="c", subcore_axis_name="s")

  scratch = (
      pltpu.VMEM((RBLK, K), jnp.int32),
      [pltpu.VMEM((K, D), jnp.float32) for _ in range(DEPTH)],
      pltpu.VMEM((RBLK, D), jnp.float32),
      pltpu.VMEM((RBLK, D), jnp.float32),
      [pltpu.SemaphoreType.DMA for _ in range(DEPTH)],
  )

  @functools.partial(
      pl.kernel,
      out_type=(
          jax.ShapeDtypeStruct((B, D), jnp.float32),
          jax.ShapeDtypeStruct((B, D), jnp.float32),
      ),
      mesh=mesh,
      compiler_params=pltpu.CompilerParams(use_tc_tiling_on_sc=False),
      scratch_types=scratch,
  )
  def sc_main(idx_hbm, emb_hbm, ctxsum_o, negsum_o,
              idx_v, bufs, ctxsum_v, negsum_v, sems):
    wid = lax.axis_index("s") * nc + lax.axis_index("c")

    def issue(b, slot):
      pltpu.make_async_copy(
          emb_hbm.at[idx_v.at[b]], bufs[slot], sems[slot]).start()

    def drain(slot):
      pltpu.make_async_copy(
          emb_hbm.at[idx_v.at[0]], bufs[slot], sems[slot]).wait()

    def seg_sum(buf, lo, hi):
      sls = [pl.ds(16 * k, 16) for k in range(NVREG)]
      acc = [buf[lo, sls[k]] for k in range(NVREG)]
      unroll = 8
      n = hi - lo - 1
      rem = n % unroll

      def step(i, a):
        j0 = lo + 1 + i * unroll
        for u in range(unroll):
          a = [a[k] + buf[j0 + u, sls[k]] for k in range(NVREG)]
        return a

      acc = lax.fori_loop(0, n // unroll, step, acc)
      for j in range(hi - rem, hi):
        acc = [acc[k] + buf[j, sls[k]] for k in range(NVREG)]
      return acc

    def reduce_store(slot, b):
      buf = bufs[slot]
      acc_c = seg_sum(buf, 0, C)
      acc_n = seg_sum(buf, C, K)
      for k in range(NVREG):
        sl = pl.ds(16 * k, 16)
        ctxsum_v[b, sl] = acc_c[k]
        negsum_v[b, sl] = acc_n[k]

    def block(blk, carry):
      base = pl.multiple_of(wid * rpw + blk * RBLK, RBLK)
      pltpu.sync_copy(idx_hbm.at[pl.ds(base, RBLK), :], idx_v)

      for s in range(DEPTH - 1):
        issue(s, s)

      def group(g, c2):
        for s in range(DEPTH):
          b = g * DEPTH + s

          @pl.when(b + DEPTH - 1 < RBLK)
          def _():
            issue(b + DEPTH - 1, (s + DEPTH - 1) % DEPTH)

          drain(s)
          reduce_store(s, b)
        return c2

      lax.fori_loop(0, RBLK // DEPTH, group, 0)

      pltpu.sync_copy(ctxsum_v, ctxsum_o.at[pl.ds(base, RBLK), :])
      pltpu.sync_copy(negsum_v, negsum_o.at[pl.ds(base, RBLK), :])
      return carry

    lax.fori_loop(0, nblk, block, 0)

  return sc_main


@functools.lru_cache(maxsize=None)
def _build_sc_center():
  info = plsc.get_sparse_core_info()
  nc, ns = info.num_cores, info.num_subcores
  nw = nc * ns
  rpw = B // nw
  ngat = rpw // 128
  mesh = plsc.VectorSubcoreMesh(core_axis_name="c", subcore_axis_name="s")

  @functools.partial(
      pl.kernel,
      out_type=jax.ShapeDtypeStruct((B, D), jnp.float32),
      mesh=mesh,
      compiler_params=pltpu.CompilerParams(use_tc_tiling_on_sc=False),
      scratch_types=(
          pltpu.VMEM((B // 32,), jnp.int32),
          pltpu.VMEM((B // 32, D), jnp.float32),
          pltpu.SemaphoreType.DMA,
      ),
  )
  def sc_center(center_hbm, emb_hbm, crows_o, cidx_v, crows_v, sem):
    wid = lax.axis_index("s") * nc + lax.axis_index("c")
    base = pl.multiple_of(wid * rpw, rpw)
    pltpu.sync_copy(center_hbm.at[pl.ds(base, rpw)], cidx_v)
    for i in range(ngat):
      pltpu.make_async_copy(
          emb_hbm.at[cidx_v.at[pl.ds(i * 128, 128)]],
          crows_v.at[pl.ds(i * 128, 128), :], sem).start()
    for i in range(ngat):
      pltpu.make_async_copy(
          emb_hbm.at[cidx_v.at[pl.ds(0, 128)]],
          crows_v.at[pl.ds(0, 128), :], sem).wait()
    pltpu.sync_copy(crows_v, crows_o.at[pl.ds(base, rpw), :])

  return sc_center


def _tc_score(crows, ctxsum, negsum):
  bt = 2048

  def body(c_ref, cs_ref, ns_ref, o_ref):
    c = c_ref[...]
    s_ctx = jnp.sum(cs_ref[...] * c, axis=1)
    s_neg = jnp.sum(ns_ref[...] * c, axis=1)
    o_ref[...] = -(jax.nn.log_sigmoid(s_ctx) + jax.nn.log_sigmoid(-s_neg))

  return pl.pallas_call(
      body,
      grid=(B // bt,),
      in_specs=[pl.BlockSpec((bt, D), lambda i: (i, 0))] * 3,
      out_specs=pl.BlockSpec((bt,), lambda i: (i,)),
      out_shape=jax.ShapeDtypeStruct((B,), jnp.float32),
  )(crows, ctxsum, negsum)


def kernel(center_word_label, context_words_labels, neg_words_labels,
           input_embedding, output_embedding):
  idx_all = _remap_idx(jnp.concatenate(
      [context_words_labels.astype(jnp.int32),
       neg_words_labels.astype(jnp.int32)], axis=1))
  out_lin = _tc_linearize(output_embedding)
  ctxsum, negsum = _build_sc_main()(idx_all, out_lin)
  crows = jnp.take(input_embedding, center_word_label, axis=0)
  return _tc_score(crows, ctxsum, negsum)


# speedup vs baseline: 1.7570x; 1.0637x over previous
import functools

import jax
import jax.numpy as jnp
from jax import lax
from jax.experimental import pallas as pl
from jax.experimental.pallas import tpu as pltpu
from jax.experimental.pallas import tpu_sc as plsc

B = 16384
D = 64
C = 20
N = 100
K = C + N
VOCAB = 1000000
RBLK = 128
NVREG = D // 16
DEPTH = 4
TRBW = 8192


HALF = TRBW // 2
NLIN = (VOCAB + TRBW - 1) // TRBW


def _tc_linearize(table):
  t_t = table.T

  def body(x_ref, o_ref):
    x = x_ref[...]
    o_ref[...] = jax.lax.concatenate([x[:, :HALF].T, x[:, HALF:].T], 1)

  out = pl.pallas_call(
      body,
      grid=(NLIN,),
      in_specs=[pl.BlockSpec((D, TRBW), lambda j: (0, j))],
      out_specs=pl.BlockSpec((HALF, 2 * D), lambda j: (j, 0)),
      out_shape=jax.ShapeDtypeStruct((NLIN * HALF, 2 * D), jnp.float32),
  )(t_t)
  return out.reshape(NLIN * TRBW, D)


def _remap_idx(r):
  j = r // TRBW
  t = r % TRBW
  return 2 * (j * HALF + (t % HALF)) + t // HALF


@functools.lru_cache(maxsize=None)
def _build_sc_main():
  info = plsc.get_sparse_core_info()
  nc, ns = info.num_cores, info.num_subcores
  nw = nc * ns
  rpw = B // nw
  nblk = rpw // RBLK
  mesh = plsc.VectorSubcoreMesh(core_axis_name="c", subcore_axis_name="s")

  scratch = (
      pltpu.VMEM((RBLK, K), jnp.int32),
      [pltpu.VMEM((K, D), jnp.float32) for _ in range(DEPTH)],
      pltpu.VMEM((RBLK, D), jnp.float32),
      pltpu.VMEM((RBLK, D), jnp.float32),
      [pltpu.SemaphoreType.DMA for _ in range(DEPTH)],
  )

  @functools.partial(
      pl.kernel,
      out_type=(
          jax.ShapeDtypeStruct((B, D), jnp.float32),
          jax.ShapeDtypeStruct((B, D), jnp.float32),
      ),
      mesh=mesh,
      compiler_params=pltpu.CompilerParams(use_tc_tiling_on_sc=False),
      scratch_types=scratch,
  )
  def sc_main(idx_hbm, emb_hbm, ctxsum_o, negsum_o,
              idx_v, bufs, ctxsum_v, negsum_v, sems):
    wid = lax.axis_index("s") * nc + lax.axis_index("c")

    def issue(b, slot):
      pltpu.make_async_copy(
          emb_hbm.at[idx_v.at[b]], bufs[slot], sems[slot]).start()

    def drain(slot):
      pltpu.make_async_copy(
          emb_hbm.at[idx_v.at[0]], bufs[slot], sems[slot]).wait()

    def seg_sum(buf, lo, hi):
      sls = [pl.ds(16 * k, 16) for k in range(NVREG)]
      acc = [buf[lo, sls[k]] for k in range(NVREG)]
      unroll = 8
      n = hi - lo - 1
      rem = n % unroll

      def step(i, a):
        j0 = lo + 1 + i * unroll
        for u in range(unroll):
          a = [a[k] + buf[j0 + u, sls[k]] for k in range(NVREG)]
        return a

      acc = lax.fori_loop(0, n // unroll, step, acc)
      for j in range(hi - rem, hi):
        acc = [acc[k] + buf[j, sls[k]] for k in range(NVREG)]
      return acc

    def reduce_store(slot, b):
      buf = bufs[slot]
      acc_c = seg_sum(buf, 0, C)
      acc_n = seg_sum(buf, C, K)
      for k in range(NVREG):
        sl = pl.ds(16 * k, 16)
        ctxsum_v[b, sl] = acc_c[k]
        negsum_v[b, sl] = acc_n[k]

    def block(blk, carry):
      base = pl.multiple_of(wid * rpw + blk * RBLK, RBLK)
      pltpu.sync_copy(idx_hbm.at[pl.ds(base, RBLK), :], idx_v)

      for s in range(DEPTH - 1):
        issue(s, s)

      def group(g, c2):
        for s in range(DEPTH):
          b = g * DEPTH + s

          @pl.when(b + DEPTH - 1 < RBLK)
          def _():
            issue(b + DEPTH - 1, (s + DEPTH - 1) % DEPTH)

          drain(s)
          reduce_store(s, b)
        return c2

      lax.fori_loop(0, RBLK // DEPTH, group, 0)

      pltpu.sync_copy(ctxsum_v, ctxsum_o.at[pl.ds(base, RBLK), :])
      pltpu.sync_copy(negsum_v, negsum_o.at[pl.ds(base, RBLK), :])
      return carry

    lax.fori_loop(0, nblk, block, 0)

  return sc_main


@functools.lru_cache(maxsize=None)
def _build_sc_center():
  info = plsc.get_sparse_core_info()
  nc, ns = info.num_cores, info.num_subcores
  nw = nc * ns
  rpw = B // nw
  ngat = rpw // 128
  mesh = plsc.VectorSubcoreMesh(core_axis_name="c", subcore_axis_name="s")

  @functools.partial(
      pl.kernel,
      out_type=jax.ShapeDtypeStruct((B, D), jnp.float32),
      mesh=mesh,
      compiler_params=pltpu.CompilerParams(use_tc_tiling_on_sc=False),
      scratch_types=(
          pltpu.VMEM((B // 32,), jnp.int32),
          pltpu.VMEM((B // 32, D), jnp.float32),
          pltpu.SemaphoreType.DMA,
      ),
  )
  def sc_center(center_hbm, emb_hbm, crows_o, cidx_v, crows_v, sem):
    wid = lax.axis_index("s") * nc + lax.axis_index("c")
    base = pl.multiple_of(wid * rpw, rpw)
    pltpu.sync_copy(center_hbm.at[pl.ds(base, rpw)], cidx_v)
    for i in range(ngat):
      pltpu.make_async_copy(
          emb_hbm.at[cidx_v.at[pl.ds(i * 128, 128)]],
          crows_v.at[pl.ds(i * 128, 128), :], sem).start()
    for i in range(ngat):
      pltpu.make_async_copy(
          emb_hbm.at[cidx_v.at[pl.ds(0, 128)]],
          crows_v.at[pl.ds(0, 128), :], sem).wait()
    pltpu.sync_copy(crows_v, crows_o.at[pl.ds(base, rpw), :])

  return sc_center


def _tc_score(crows, ctxsum, negsum):
  bt = 2048

  def body(c_ref, cs_ref, ns_ref, o_ref):
    c = c_ref[...]
    s_ctx = jnp.sum(cs_ref[...] * c, axis=1)
    s_neg = jnp.sum(ns_ref[...] * c, axis=1)
    o_ref[...] = -(jax.nn.log_sigmoid(s_ctx) + jax.nn.log_sigmoid(-s_neg))

  return pl.pallas_call(
      body,
      grid=(B // bt,),
      in_specs=[pl.BlockSpec((bt, D), lambda i: (i, 0))] * 3,
      out_specs=pl.BlockSpec((bt,), lambda i: (i,)),
      out_shape=jax.ShapeDtypeStruct((B,), jnp.float32),
  )(crows, ctxsum, negsum)


def kernel(center_word_label, context_words_labels, neg_words_labels,
           input_embedding, output_embedding):
  idx_all = _remap_idx(jnp.concatenate(
      [context_words_labels.astype(jnp.int32),
       neg_words_labels.astype(jnp.int32)], axis=1))
  idx_all, out_emb = jax.lax.optimization_barrier(
      (idx_all, output_embedding))
  out_lin = _tc_linearize(out_emb)
  ctxsum, negsum = _build_sc_main()(idx_all, out_lin)
  crows = jnp.take(input_embedding, center_word_label, axis=0)
  return _tc_score(crows, ctxsum, negsum)


# speedup vs baseline: 1.8677x; 1.0630x over previous
import functools

import jax
import jax.numpy as jnp
from jax import lax
from jax.experimental import pallas as pl
from jax.experimental.pallas import tpu as pltpu
from jax.experimental.pallas import tpu_sc as plsc

B = 16384
D = 64
C = 20
N = 100
K = C + N
VOCAB = 1000000
RBLK = 128
NVREG = D // 16
DEPTH = 4
TRBW = 32768


HALF = TRBW // 2
NLIN = (VOCAB + TRBW - 1) // TRBW


def _tc_linearize(table):
  t_t = table.T

  def body(x_ref, o_ref):
    x = x_ref[...]
    o_ref[...] = jax.lax.concatenate([x[:, :HALF].T, x[:, HALF:].T], 1)

  out = pl.pallas_call(
      body,
      grid=(NLIN,),
      in_specs=[pl.BlockSpec((D, TRBW), lambda j: (0, j))],
      out_specs=pl.BlockSpec((HALF, 2 * D), lambda j: (j, 0)),
      out_shape=jax.ShapeDtypeStruct((NLIN * HALF, 2 * D), jnp.float32),
  )(t_t)
  return out.reshape(NLIN * TRBW, D)


def _remap_idx(r):
  j = r // TRBW
  t = r % TRBW
  return 2 * (j * HALF + (t % HALF)) + t // HALF


@functools.lru_cache(maxsize=None)
def _build_sc_main():
  info = plsc.get_sparse_core_info()
  nc, ns = info.num_cores, info.num_subcores
  nw = nc * ns
  rpw = B // nw
  nblk = rpw // RBLK
  mesh = plsc.VectorSubcoreMesh(core_axis_name="c", subcore_axis_name="s")

  scratch = (
      pltpu.VMEM((RBLK, K), jnp.int32),
      [pltpu.VMEM((K, D), jnp.float32) for _ in range(DEPTH)],
      pltpu.VMEM((RBLK, D), jnp.float32),
      pltpu.VMEM((RBLK, D), jnp.float32),
      [pltpu.SemaphoreType.DMA for _ in range(DEPTH)],
  )

  @functools.partial(
      pl.kernel,
      out_type=(
          jax.ShapeDtypeStruct((B, D), jnp.float32),
          jax.ShapeDtypeStruct((B, D), jnp.float32),
      ),
      mesh=mesh,
      compiler_params=pltpu.CompilerParams(use_tc_tiling_on_sc=False),
      scratch_types=scratch,
  )
  def sc_main(idx_hbm, emb_hbm, ctxsum_o, negsum_o,
              idx_v, bufs, ctxsum_v, negsum_v, sems):
    wid = lax.axis_index("s") * nc + lax.axis_index("c")

    def issue(b, slot):
      pltpu.make_async_copy(
          emb_hbm.at[idx_v.at[b]], bufs[slot], sems[slot]).start()

    def drain(slot):
      pltpu.make_async_copy(
          emb_hbm.at[idx_v.at[0]], bufs[slot], sems[slot]).wait()

    def seg_sum(buf, lo, hi):
      sls = [pl.ds(16 * k, 16) for k in range(NVREG)]
      acc = [buf[lo, sls[k]] for k in range(NVREG)]
      unroll = 8
      n = hi - lo - 1
      rem = n % unroll

      def step(i, a):
        j0 = lo + 1 + i * unroll
        for u in range(unroll):
          a = [a[k] + buf[j0 + u, sls[k]] for k in range(NVREG)]
        return a

      acc = lax.fori_loop(0, n // unroll, step, acc)
      for j in range(hi - rem, hi):
        acc = [acc[k] + buf[j, sls[k]] for k in range(NVREG)]
      return acc

    def reduce_store(slot, b):
      buf = bufs[slot]
      acc_c = seg_sum(buf, 0, C)
      acc_n = seg_sum(buf, C, K)
      for k in range(NVREG):
        sl = pl.ds(16 * k, 16)
        ctxsum_v[b, sl] = acc_c[k]
        negsum_v[b, sl] = acc_n[k]

    def block(blk, carry):
      base = pl.multiple_of(wid * rpw + blk * RBLK, RBLK)
      pltpu.sync_copy(idx_hbm.at[pl.ds(base, RBLK), :], idx_v)

      for s in range(DEPTH - 1):
        issue(s, s)

      def group(g, c2):
        for s in range(DEPTH):
          b = g * DEPTH + s

          @pl.when(b + DEPTH - 1 < RBLK)
          def _():
            issue(b + DEPTH - 1, (s + DEPTH - 1) % DEPTH)

          drain(s)
          reduce_store(s, b)
        return c2

      lax.fori_loop(0, RBLK // DEPTH, group, 0)

      pltpu.sync_copy(ctxsum_v, ctxsum_o.at[pl.ds(base, RBLK), :])
      pltpu.sync_copy(negsum_v, negsum_o.at[pl.ds(base, RBLK), :])
      return carry

    lax.fori_loop(0, nblk, block, 0)

  return sc_main


@functools.lru_cache(maxsize=None)
def _build_sc_center():
  info = plsc.get_sparse_core_info()
  nc, ns = info.num_cores, info.num_subcores
  nw = nc * ns
  rpw = B // nw
  ngat = rpw // 128
  mesh = plsc.VectorSubcoreMesh(core_axis_name="c", subcore_axis_name="s")

  @functools.partial(
      pl.kernel,
      out_type=jax.ShapeDtypeStruct((B, D), jnp.float32),
      mesh=mesh,
      compiler_params=pltpu.CompilerParams(use_tc_tiling_on_sc=False),
      scratch_types=(
          pltpu.VMEM((B // 32,), jnp.int32),
          pltpu.VMEM((B // 32, D), jnp.float32),
          pltpu.SemaphoreType.DMA,
      ),
  )
  def sc_center(center_hbm, emb_hbm, crows_o, cidx_v, crows_v, sem):
    wid = lax.axis_index("s") * nc + lax.axis_index("c")
    base = pl.multiple_of(wid * rpw, rpw)
    pltpu.sync_copy(center_hbm.at[pl.ds(base, rpw)], cidx_v)
    for i in range(ngat):
      pltpu.make_async_copy(
          emb_hbm.at[cidx_v.at[pl.ds(i * 128, 128)]],
          crows_v.at[pl.ds(i * 128, 128), :], sem).start()
    for i in range(ngat):
      pltpu.make_async_copy(
          emb_hbm.at[cidx_v.at[pl.ds(0, 128)]],
          crows_v.at[pl.ds(0, 128), :], sem).wait()
    pltpu.sync_copy(crows_v, crows_o.at[pl.ds(base, rpw), :])

  return sc_center


def _tc_score(crows, ctxsum, negsum):
  bt = 2048

  def body(c_ref, cs_ref, ns_ref, o_ref):
    c = c_ref[...]
    s_ctx = jnp.sum(cs_ref[...] * c, axis=1)
    s_neg = jnp.sum(ns_ref[...] * c, axis=1)
    o_ref[...] = -(jax.nn.log_sigmoid(s_ctx) + jax.nn.log_sigmoid(-s_neg))

  return pl.pallas_call(
      body,
      grid=(B // bt,),
      in_specs=[pl.BlockSpec((bt, D), lambda i: (i, 0))] * 3,
      out_specs=pl.BlockSpec((bt,), lambda i: (i,)),
      out_shape=jax.ShapeDtypeStruct((B,), jnp.float32),
  )(crows, ctxsum, negsum)


def kernel(center_word_label, context_words_labels, neg_words_labels,
           input_embedding, output_embedding):
  idx_all = _remap_idx(jnp.concatenate(
      [context_words_labels.astype(jnp.int32),
       neg_words_labels.astype(jnp.int32)], axis=1))
  idx_all, out_emb = jax.lax.optimization_barrier(
      (idx_all, output_embedding))
  out_lin = _tc_linearize(out_emb)
  ctxsum, negsum = _build_sc_main()(idx_all, out_lin)
  crows = jnp.take(input_embedding, center_word_label, axis=0)
  return _tc_score(crows, ctxsum, negsum)
